# baseline (speedup 1.0000x reference)
import numpy as np
import jax
import jax.numpy as jnp
from jax import lax
from jax.experimental import pallas as pl
from jax.experimental.pallas import tpu as pltpu

_N = 4096
_D = 32
_K = 256
_NSUB = 4
_ALPHA = 0.5
_EPS = 1e-12

_RB = 256
_NB = _N // _RB

_BIG = 3.0e38
_HK = _K // 2
_SROWS = 2 * _NSUB * _K


def _subset_idx() -> np.ndarray:
    rows = []
    for s in range(_NSUB):
        rs = np.random.RandomState(s)
        rows.append(np.sort(rs.choice(_N, size=_K, replace=False)))
    return np.asarray(rows, dtype=np.int32)


_IDX = _subset_idx()


def _dot(a, b, dims):
    return lax.dot_general(a, b, (dims, ((), ())),
                           preferred_element_type=jnp.float32,
                           precision=lax.Precision.HIGHEST)


def _metric_body(dist_ref, mask_ref, emb_ref, embb_ref, out_ref):
    i = pl.program_id(0)
    e = emb_ref[...]
    eb = embb_ref[...]
    r_full = jnp.sum(e * e, axis=1)
    r_blk = jnp.sum(eb * eb, axis=1)
    ones_b = jnp.ones((_RB, 1), jnp.float32)
    ones_f = jnp.ones((_N, 1), jnp.float32)
    aug_a = jnp.concatenate([eb, ones_b, r_blk[:, None]], axis=1)
    aug_b = jnp.concatenate([-2.0 * e, r_full[:, None], ones_f], axis=1)
    d2 = _dot(aug_a, aug_b, ((1,), (1,)))
    emb_dist = jnp.sqrt(jnp.maximum(d2, _EPS))
    part = jnp.sum(mask_ref[...] * (dist_ref[...] - emb_dist) ** 2)

    @pl.when(i == 0)
    def _():
        out_ref[0, 0] = 0.0

    out_ref[0, 0] += part


def _metric_call(embedding, dist_mat, edge_mask):
    return pl.pallas_call(
        _metric_body,
        grid=(_NB,),
        in_specs=[
            pl.BlockSpec((_RB, _N), lambda i: (i, 0)),
            pl.BlockSpec((_RB, _N), lambda i: (i, 0)),
            pl.BlockSpec((_N, _D), lambda i: (0, 0)),
            pl.BlockSpec((_RB, _D), lambda i: (i, 0)),
        ],
        out_specs=pl.BlockSpec(memory_space=pltpu.SMEM),
        out_shape=jax.ShapeDtypeStruct((1, 1), jnp.float32),
        compiler_params=pltpu.CompilerParams(
            dimension_semantics=("arbitrary",)),
    )(dist_mat, edge_mask, embedding, embedding)


def _roll(a, shift, axis):
    n = a.shape[axis]
    shift %= n
    if shift == 0:
        return a
    lo = lax.slice_in_dim(a, n - shift, n, axis=axis)
    hi = lax.slice_in_dim(a, 0, n - shift, axis=axis)
    return lax.concatenate([lo, hi], dimension=axis)


def _cdist_gram(x):
    r = jnp.sum(x * x, axis=1)
    g = _dot(x, x, ((1,), (1,)))
    d2 = r[:, None] + r[None, :] - 2.0 * g
    return jnp.sqrt(jnp.maximum(d2, _EPS))


def _topo_body(dist_ref, emb_ref, idxv_ref, out_ref, rows_v, sem):
    copies = []
    for s in range(_NSUB):
        for r in range(_K):
            c = pltpu.make_async_copy(dist_ref.at[int(_IDX[s, r])],
                                      rows_v.at[s, r], sem.at[s])
            c.start()
            copies.append(c)

    e = emb_ref[...]
    iota = lax.broadcasted_iota(jnp.int32, (_N, _K), 0)
    codoms = []
    onehots = []
    for s in range(_NSUB):
        cols = idxv_ref[s, 0, :]
        onehot = (iota == cols[None, :]).astype(jnp.float32)
        onehots.append(onehot)
        sub_e = _dot(onehot, e, ((0,), (0,)))
        codoms.append(_cdist_gram(sub_e))

    for c in copies:
        c.wait()

    doms = [_dot(rows_v[s], onehots[s], ((1,), (0,))) for s in range(_NSUB)]

    a = jnp.concatenate(doms + codoms, axis=0)

    col_i = lax.broadcasted_iota(jnp.int32, (1, _K), 1)
    row_i = lax.broadcasted_iota(jnp.int32, (_SROWS, 1), 0) & (_K - 1)

    def bit(x, b):
        return (x >> b) & 1

    for b in range(8):
        sh = 1 << b
        rolled = _roll(a, -sh, 1)
        a = jnp.where(bit(row_i, b) == 1, rolled, a)

    col0 = jnp.where(row_i < _HK, a[:, _HK:_HK + 1], _BIG)
    a = jnp.concatenate([col0, a[:, 1:_HK]], axis=1)

    col_i = lax.broadcasted_iota(jnp.int32, (1, _HK), 1)

    for ke in range(1, 16):
        for bj in reversed(range(ke)):
            j = 1 << bj
            if bj < 7:
                up = _roll(a, -j, 1)
                dn = _roll(a, j, 1)
                bitj = bit(col_i, bj)
            else:
                sh = j >> 7
                up = _roll(a, -sh, 0)
                dn = _roll(a, sh, 0)
                bitj = bit(row_i, bj - 7)
            partner = jnp.where(bitj == 0, up, dn)
            if ke == 15:
                keep_min = bitj == 0
            else:
                ascbit = bit(col_i, ke) if ke < 7 else bit(row_i, ke - 7)
                keep_min = (bitj ^ ascbit) == 0
            a = jnp.where(keep_min, jnp.minimum(a, partner),
                          jnp.maximum(a, partner))

    diff = a[:_NSUB * _K, :] - a[_NSUB * _K:, :]
    out_ref[0, 0] = 2.0 * jnp.sum(diff * diff)


def _topo_call(dist_mat, embedding):
    idx_vec = jnp.asarray(_IDX).reshape(_NSUB, 1, _K)
    return pl.pallas_call(
        _topo_body,
        in_specs=[
            pl.BlockSpec(memory_space=pltpu.MemorySpace.HBM),
            pl.BlockSpec((_N, _D), lambda: (0, 0)),
            pl.BlockSpec((_NSUB, 1, _K), lambda: (0, 0, 0)),
        ],
        out_specs=pl.BlockSpec(memory_space=pltpu.SMEM),
        out_shape=jax.ShapeDtypeStruct((1, 1), jnp.float32),
        scratch_shapes=[
            pltpu.VMEM((_NSUB, _K, _N), jnp.float32),
            pltpu.SemaphoreType.DMA((_NSUB,)),
        ],
    )(dist_mat, embedding, idx_vec)


@jax.jit
def kernel(embedding, dist_mat, edge_mask):
    metric = _metric_call(embedding, dist_mat, edge_mask)[0, 0]
    topo = _topo_call(dist_mat, embedding)[0, 0] / _NSUB
    return (_ALPHA * metric + (1.0 - _ALPHA) * topo).astype(jnp.float32)
